# baseline (speedup 1.0000x reference)
import functools

import jax
import jax.numpy as jnp
from jax import lax
from jax.experimental import pallas as pl
from jax.experimental.pallas import tpu as pltpu
from jax.experimental.pallas import tpu_sc as plsc

_NC, _NS, _L = 2, 16, 16
_NW = _NC * _NS
_CH = 128
_CP = pltpu.CompilerParams(use_tc_tiling_on_sc=False)


def _lane_bcast(vec, lane):
    idx = jnp.full((16, 1), lane, jnp.int32)
    dn = lax.GatherDimensionNumbers(
        offset_dims=(), collapsed_slice_dims=(0,), start_index_map=(0,))
    return lax.gather(vec, idx, dn, (1,),
                      mode=lax.GatherScatterMode.PROMISE_IN_BOUNDS)


def _make_deg_pass(NPAD, EPT_CHUNKS):
    EPT = EPT_CHUNKS * _CH
    RPT = NPAD // _NS
    mesh = plsc.VectorSubcoreMesh(core_axis_name="c", subcore_axis_name="s")

    @functools.partial(
        pl.kernel,
        out_type=jax.ShapeDtypeStruct((_NC, NPAD, 16), jnp.float32),
        mesh=mesh,
        compiler_params=_CP,
        scratch_types=[
            pltpu.VMEM((_CH,), jnp.int32),
            pltpu.VMEM((_CH, 16), jnp.float32),
            pltpu.VMEM_SHARED((NPAD, 16), jnp.float32),
        ],
    )
    def deg_pass(dst_hbm, out_hbm, idx_d, ones, acc):
        c = lax.axis_index("c")
        s = lax.axis_index("s")
        tid = c * _NS + s
        zero = jnp.zeros((16,), jnp.float32)
        one = jnp.ones((16,), jnp.float32)

        def zrow(r, _):
            ones[r, pl.ds(0, 16)] = zero
            return 0
        lax.fori_loop(0, _CH, zrow, 0)
        for k in range(RPT // _CH):
            pltpu.sync_copy(ones, acc.at[pl.ds(s * RPT + k * _CH, _CH)])

        def orow(r, _):
            ones[r, pl.ds(0, 16)] = one
            return 0
        lax.fori_loop(0, _CH, orow, 0)
        plsc.subcore_barrier()

        def chunk(ci, _):
            base = tid * EPT + ci * _CH
            pltpu.sync_copy(dst_hbm.at[pl.ds(base, _CH)], idx_d)
            pltpu.sync_copy(ones, acc.at[idx_d], add=True)
            return 0
        lax.fori_loop(0, EPT_CHUNKS, chunk, 0)
        plsc.subcore_barrier()
        for k in range(RPT // _CH):
            r0 = s * RPT + k * _CH
            pltpu.sync_copy(acc.at[pl.ds(r0, _CH)], ones)
            pltpu.sync_copy(ones, out_hbm.at[c, pl.ds(r0, _CH)])

    return deg_pass


_NB = 4


def _make_gcn_pass(NPAD, W, EPT_CHUNKS):
    RPT = NPAD // _NS
    mesh = plsc.VectorSubcoreMesh(core_axis_name="c", subcore_axis_name="s")

    @functools.partial(
        pl.kernel,
        out_type=jax.ShapeDtypeStruct((_NC, NPAD, W), jnp.float32),
        mesh=mesh,
        compiler_params=_CP,
        scratch_types=[
            pltpu.VMEM((EPT_CHUNKS, _CH), jnp.int32),
            pltpu.VMEM((EPT_CHUNKS, _CH), jnp.int32),
            [pltpu.VMEM((_CH, W), jnp.float32)] * _NB,
            pltpu.VMEM_SHARED((NPAD, W), jnp.float32),
            [pltpu.SemaphoreType.DMA] * _NB,
            [pltpu.SemaphoreType.DMA] * _NB,
        ],
    )
    def gcn_pass(table_hbm, src_hbm, dst_hbm, out_hbm, isrc, idst, rows, acc,
                 gsem, ssem):
        c = lax.axis_index("c")
        s = lax.axis_index("s")
        tid = c * _NS + s
        zero = jnp.zeros((16,), jnp.float32)

        def zrow(r, _):
            for k in range(W // 16):
                rows[0][r, pl.ds(k * 16, 16)] = zero
            return 0
        lax.fori_loop(0, _CH, zrow, 0)
        for k in range(RPT // _CH):
            pltpu.sync_copy(rows[0], acc.at[pl.ds(s * RPT + k * _CH, _CH)])
        pltpu.sync_copy(src_hbm.at[tid], isrc)
        pltpu.sync_copy(dst_hbm.at[tid], idst)
        plsc.subcore_barrier()

        gets = {}
        for b in range(min(_NB, EPT_CHUNKS)):
            gets[b] = pltpu.async_copy(table_hbm.at[isrc.at[b]], rows[b],
                                       gsem[b])
        puts = [None] * _NB
        for ci in range(EPT_CHUNKS):
            b = ci % _NB
            gets.pop(ci).wait()
            puts[b] = pltpu.async_copy(rows[b], acc.at[idst.at[ci]], ssem[b],
                                       add=True)
            nxt = ci + _NB
            if nxt < EPT_CHUNKS:
                puts[b].wait()
                puts[b] = None
                gets[nxt] = pltpu.async_copy(table_hbm.at[isrc.at[nxt]],
                                             rows[b], gsem[b])
        for b in range(_NB):
            if puts[b] is not None:
                puts[b].wait()
        plsc.subcore_barrier()
        for k in range(RPT // _CH):
            r0 = s * RPT + k * _CH
            pltpu.sync_copy(acc.at[pl.ds(r0, _CH)], rows[0])
            pltpu.sync_copy(rows[0], out_hbm.at[c, pl.ds(r0, _CH)])

    return gcn_pass


_NBG = 3


def _make_gat_pass(NPAD, EPT_CHUNKS):
    RPT = NPAD // _NS
    mesh = plsc.VectorSubcoreMesh(core_axis_name="c", subcore_axis_name="s")

    @functools.partial(
        pl.kernel,
        out_type=(
            jax.ShapeDtypeStruct((_NC, NPAD, 64), jnp.float32),
            jax.ShapeDtypeStruct((_NC, NPAD, 16), jnp.float32),
        ),
        mesh=mesh,
        compiler_params=_CP,
        scratch_types=[
            pltpu.VMEM((EPT_CHUNKS, _CH), jnp.int32),
            pltpu.VMEM((EPT_CHUNKS, _CH), jnp.int32),
            [pltpu.VMEM((_CH, 64), jnp.float32)] * _NBG,
            [pltpu.VMEM((_CH, 16), jnp.float32)] * _NBG,
            [pltpu.VMEM((_CH, 16), jnp.float32)] * _NBG,
            [pltpu.VMEM((_CH, 16), jnp.float32)] * _NBG,
            pltpu.VMEM_SHARED((NPAD, 64), jnp.float32),
            pltpu.VMEM_SHARED((NPAD, 16), jnp.float32),
            [pltpu.SemaphoreType.DMA] * _NBG,
            [pltpu.SemaphoreType.DMA] * _NBG,
            [pltpu.SemaphoreType.DMA] * _NBG,
            [pltpu.SemaphoreType.DMA] * _NBG,
        ],
    )
    def gat_pass(table_hbm, als_hbm, ald_hbm, src_hbm, dst_hbm, num_hbm, den_hbm,
                 isrc, idst, rows, ars, ard, wbuf, num_acc, den_acc,
                 rsem, asem, nsem, dsem):
        c = lax.axis_index("c")
        s = lax.axis_index("s")
        tid = c * _NS + s
        zero = jnp.zeros((16,), jnp.float32)

        def zrow(r, _):
            for k in range(4):
                rows[0][r, pl.ds(k * 16, 16)] = zero
            wbuf[0][r, pl.ds(0, 16)] = zero
            return 0
        lax.fori_loop(0, _CH, zrow, 0)
        for k in range(RPT // _CH):
            r0 = s * RPT + k * _CH
            pltpu.sync_copy(rows[0], num_acc.at[pl.ds(r0, _CH)])
            pltpu.sync_copy(wbuf[0], den_acc.at[pl.ds(r0, _CH)])
        pltpu.sync_copy(src_hbm.at[tid], isrc)
        pltpu.sync_copy(dst_hbm.at[tid], idst)
        plsc.subcore_barrier()

        def gathers(ci, b):
            return (
                pltpu.async_copy(table_hbm.at[isrc.at[ci]], rows[b], rsem[b]),
                pltpu.async_copy(als_hbm.at[isrc.at[ci]], ars[b], asem[b]),
                pltpu.async_copy(ald_hbm.at[idst.at[ci]], ard[b], asem[b]),
            )

        gets = {}
        for b in range(min(_NBG, EPT_CHUNKS)):
            gets[b] = gathers(b, b)
        putsN = [None] * _NBG
        putsD = [None] * _NBG
        for ci in range(EPT_CHUNKS):
            b = ci % _NBG
            g1, g2, g3 = gets.pop(ci)
            g2.wait()
            g3.wait()
            g1.wait()
            rows_b, ars_b, ard_b, wbuf_b = rows[b], ars[b], ard[b], wbuf[b]

            def edge(r, _):
                ev = ars_b[r, pl.ds(0, 16)] + ard_b[r, pl.ds(0, 16)]
                ev = jnp.where(ev > 0, ev, 0.2 * ev)
                wv = jnp.exp(ev)
                wbuf_b[r, pl.ds(0, 16)] = wv
                for h in range(4):
                    bb = _lane_bcast(wv, h)
                    rows_b[r, pl.ds(h * 16, 16)] = rows_b[r, pl.ds(h * 16, 16)] * bb
                return 0
            lax.fori_loop(0, _CH, edge, 0)
            putsN[b] = pltpu.async_copy(rows[b], num_acc.at[idst.at[ci]],
                                        nsem[b], add=True)
            putsD[b] = pltpu.async_copy(wbuf[b], den_acc.at[idst.at[ci]],
                                        dsem[b], add=True)
            nxt = ci + _NBG
            if nxt < EPT_CHUNKS:
                putsN[b].wait()
                putsD[b].wait()
                putsN[b] = putsD[b] = None
                gets[nxt] = gathers(nxt, b)
        for b in range(_NBG):
            if putsN[b] is not None:
                putsN[b].wait()
                putsD[b].wait()
        plsc.subcore_barrier()
        for k in range(RPT // _CH):
            r0 = s * RPT + k * _CH
            pltpu.sync_copy(num_acc.at[pl.ds(r0, _CH)], rows[0])
            pltpu.sync_copy(rows[0], num_hbm.at[c, pl.ds(r0, _CH)])
            pltpu.sync_copy(den_acc.at[pl.ds(r0, _CH)], wbuf[0])
            pltpu.sync_copy(wbuf[0], den_hbm.at[c, pl.ds(r0, _CH)])

    return gat_pass


_BR = 2000


def _tc_layer1(x, W1, d0, d1):
    N, Din = x.shape
    Hid = W1.shape[1]
    grid = N // _BR

    def body(x_ref, w_ref, d0_ref, d1_ref, t_ref, dis_ref):
        deg = d0_ref[...] + d1_ref[...] + 1.0
        dis = lax.rsqrt(deg)
        dis_ref[...] = dis
        t_ref[...] = jnp.dot(x_ref[...], w_ref[...],
                             preferred_element_type=jnp.float32) * dis

    return pl.pallas_call(
        body,
        grid=(grid,),
        in_specs=[
            pl.BlockSpec((_BR, Din), lambda i: (i, 0)),
            pl.BlockSpec((Din, Hid), lambda i: (0, 0)),
            pl.BlockSpec((_BR, 1), lambda i: (i, 0)),
            pl.BlockSpec((_BR, 1), lambda i: (i, 0)),
        ],
        out_specs=[
            pl.BlockSpec((_BR, Hid), lambda i: (i, 0)),
            pl.BlockSpec((_BR, 1), lambda i: (i, 0)),
        ],
        out_shape=[
            jax.ShapeDtypeStruct((N, Hid), jnp.float32),
            jax.ShapeDtypeStruct((N, 1), jnp.float32),
        ],
    )(x, W1, d0, d1)


def _tc_mid(p0, p1, t, dis, b, Wn):
    N, W = t.shape
    W2 = Wn.shape[1]
    grid = N // _BR

    def body(p0_ref, p1_ref, t_ref, dis_ref, b_ref, w_ref, out_ref):
        dis = dis_ref[...]
        h = dis * (p0_ref[...] + p1_ref[...] + t_ref[...]) + b_ref[...]
        h = jnp.maximum(h, 0.0)
        out_ref[...] = jnp.dot(h, w_ref[...],
                               preferred_element_type=jnp.float32) * dis

    return pl.pallas_call(
        body,
        grid=(grid,),
        in_specs=[
            pl.BlockSpec((_BR, W), lambda i: (i, 0)),
            pl.BlockSpec((_BR, W), lambda i: (i, 0)),
            pl.BlockSpec((_BR, W), lambda i: (i, 0)),
            pl.BlockSpec((_BR, 1), lambda i: (i, 0)),
            pl.BlockSpec((1, W), lambda i: (0, 0)),
            pl.BlockSpec((W, W2), lambda i: (0, 0)),
        ],
        out_specs=pl.BlockSpec((_BR, W2), lambda i: (i, 0)),
        out_shape=jax.ShapeDtypeStruct((N, W2), jnp.float32),
    )(p0, p1, t, dis, b, Wn)


def _tc_mid3(p0, p1, t, dis, b3, Wa, a_sf, a_df):
    N, W = t.shape
    HC = Wa.shape[1]
    grid = N // _BR

    def body(p0_ref, p1_ref, t_ref, dis_ref, b_ref, wa_ref, as_ref, ad_ref,
             hg_ref, als_ref, ald_ref, ws_ref):
        dis = dis_ref[...]
        h3 = dis * (p0_ref[...] + p1_ref[...] + t_ref[...]) + b_ref[...]
        h3 = jnp.maximum(h3, 0.0)
        hg = jnp.dot(h3, wa_ref[...], preferred_element_type=jnp.float32)
        hg_ref[...] = hg
        al_s = (hg * as_ref[...]).reshape(_BR, 4, 16).sum(-1)
        al_d = (hg * ad_ref[...]).reshape(_BR, 4, 16).sum(-1)
        z12 = jnp.zeros((_BR, 12), jnp.float32)
        als_ref[...] = jnp.concatenate([al_s, z12], axis=1)
        ald_ref[...] = jnp.concatenate([al_d, z12], axis=1)
        es = al_s + al_d
        es = jnp.where(es > 0, es, 0.2 * es)
        ws_ref[...] = jnp.exp(es)

    return pl.pallas_call(
        body,
        grid=(grid,),
        in_specs=[
            pl.BlockSpec((_BR, W), lambda i: (i, 0)),
            pl.BlockSpec((_BR, W), lambda i: (i, 0)),
            pl.BlockSpec((_BR, W), lambda i: (i, 0)),
            pl.BlockSpec((_BR, 1), lambda i: (i, 0)),
            pl.BlockSpec((1, W), lambda i: (0, 0)),
            pl.BlockSpec((W, HC), lambda i: (0, 0)),
            pl.BlockSpec((1, HC), lambda i: (0, 0)),
            pl.BlockSpec((1, HC), lambda i: (0, 0)),
        ],
        out_specs=[
            pl.BlockSpec((_BR, HC), lambda i: (i, 0)),
            pl.BlockSpec((_BR, 16), lambda i: (i, 0)),
            pl.BlockSpec((_BR, 16), lambda i: (i, 0)),
            pl.BlockSpec((_BR, 4), lambda i: (i, 0)),
        ],
        out_shape=[
            jax.ShapeDtypeStruct((N, HC), jnp.float32),
            jax.ShapeDtypeStruct((N, 16), jnp.float32),
            jax.ShapeDtypeStruct((N, 16), jnp.float32),
            jax.ShapeDtypeStruct((N, 4), jnp.float32),
        ],
    )(p0, p1, t, dis, b3, Wa, a_sf, a_df)


def _tc_final(np0, np1, d0, d1, hg, ws, b_att, n_nodes, Wc1, bc1p, Wc2p, bc2p):
    N, HC = hg.shape
    grid = N // _BR

    def body(np0_ref, np1_ref, d0_ref, d1_ref, hg_ref, ws_ref, b_ref,
             w1_ref, b1_ref, w2_ref, b2_ref, acc_ref, out_ref):
        i = pl.program_id(0)
        ws = ws_ref[...]
        wrep = jnp.broadcast_to(ws[:, :, None], (_BR, 4, 16)).reshape(_BR, 64)
        num = np0_ref[...] + np1_ref[...] + wrep * hg_ref[...]
        den4 = d0_ref[...][:, :4] + d1_ref[...][:, :4] + ws
        denrep = jnp.broadcast_to(den4[:, :, None], (_BR, 4, 16)).reshape(_BR, 64)
        h_att = jnp.maximum(num / (denrep + 1e-16) + b_ref[...], 0.0)
        part = jnp.sum(h_att, axis=0, keepdims=True)

        @pl.when(i == 0)
        def _():
            acc_ref[...] = jnp.zeros_like(acc_ref)
        acc_ref[...] += part

        @pl.when(i == grid - 1)
        def _():
            pooled = acc_ref[...] * (1.0 / n_nodes)
            z = jnp.maximum(jnp.dot(pooled, w1_ref[...],
                                    preferred_element_type=jnp.float32)
                            + b1_ref[...], 0.0)
            out_ref[...] = jnp.dot(z, w2_ref[...],
                                   preferred_element_type=jnp.float32) + b2_ref[...]

    return pl.pallas_call(
        body,
        grid=(grid,),
        in_specs=[
            pl.BlockSpec((_BR, HC), lambda i: (i, 0)),
            pl.BlockSpec((_BR, HC), lambda i: (i, 0)),
            pl.BlockSpec((_BR, 16), lambda i: (i, 0)),
            pl.BlockSpec((_BR, 16), lambda i: (i, 0)),
            pl.BlockSpec((_BR, HC), lambda i: (i, 0)),
            pl.BlockSpec((_BR, 4), lambda i: (i, 0)),
            pl.BlockSpec((1, HC), lambda i: (0, 0)),
            pl.BlockSpec(Wc1.shape, lambda i: (0, 0)),
            pl.BlockSpec(bc1p.shape, lambda i: (0, 0)),
            pl.BlockSpec(Wc2p.shape, lambda i: (0, 0)),
            pl.BlockSpec(bc2p.shape, lambda i: (0, 0)),
        ],
        out_specs=[
            pl.BlockSpec((1, HC), lambda i: (0, 0)),
            pl.BlockSpec((1, 128), lambda i: (0, 0)),
        ],
        out_shape=[
            jax.ShapeDtypeStruct((1, HC), jnp.float32),
            jax.ShapeDtypeStruct((1, 128), jnp.float32),
        ],
    )(np0, np1, d0, d1, hg, ws, b_att, Wc1, bc1p, Wc2p, bc2p)


def kernel(x, edge_index, W1, b1, W2, b2, W3, b3, Wa, a_src, a_dst, b_att,
           Wc1, bc1, Wc2, bc2):
    N = x.shape[0]
    E = edge_index.shape[1]
    NPAD = 10240
    EPT_CHUNKS = -(-E // (_NW * _CH))
    EPAD = EPT_CHUNKS * _CH * _NW

    src = edge_index[0]
    dst = edge_index[1]
    pad_idx = jnp.full((EPAD - E,), N, jnp.int32)
    srcp = jnp.concatenate([src, pad_idx])
    dstp = jnp.concatenate([dst, pad_idx])
    srcp3 = srcp.reshape(_NW, EPT_CHUNKS, _CH)
    dstp3 = dstp.reshape(_NW, EPT_CHUNKS, _CH)

    def padN(a):
        return jnp.pad(a, ((0, NPAD - N), (0, 0)))

    degp = _make_deg_pass(NPAD, EPT_CHUNKS)(dstp)
    d0 = degp[0, :N, 0:1]
    d1 = degp[1, :N, 0:1]

    t1, dis = _tc_layer1(x, W1, d0, d1)
    p = _make_gcn_pass(NPAD, 64, EPT_CHUNKS)(padN(t1), srcp3, dstp3)
    t2 = _tc_mid(p[0, :N], p[1, :N], t1, dis, b1.reshape(1, -1), W2)
    p = _make_gcn_pass(NPAD, 64, EPT_CHUNKS)(padN(t2), srcp3, dstp3)
    t3 = _tc_mid(p[0, :N], p[1, :N], t2, dis, b2.reshape(1, -1), W3)
    p = _make_gcn_pass(NPAD, 32, EPT_CHUNKS)(padN(t3), srcp3, dstp3)

    a_sf = a_src.reshape(1, -1)
    a_df = a_dst.reshape(1, -1)
    hg, als16, ald16, ws = _tc_mid3(p[0, :N], p[1, :N], t3, dis,
                                    b3.reshape(1, -1), Wa, a_sf, a_df)

    nump, denp = _make_gat_pass(NPAD, EPT_CHUNKS)(
        padN(hg), padN(als16), padN(ald16), srcp3, dstp3)

    bc1p = bc1.reshape(1, -1)
    Wc2p = jnp.pad(Wc2, ((0, 0), (0, 126)))
    bc2p = jnp.pad(bc2.reshape(1, -1), ((0, 0), (0, 126)))
    _, outp = _tc_final(nump[0, :N], nump[1, :N], denp[0, :N], denp[1, :N],
                        hg, ws, b_att.reshape(1, -1), float(N),
                        Wc1, bc1p, Wc2p, bc2p)
    return outp[:, :2]


# speedup vs baseline: 1.0446x; 1.0446x over previous
import functools

import jax
import jax.numpy as jnp
from jax import lax
from jax.experimental import pallas as pl
from jax.experimental.pallas import tpu as pltpu
from jax.experimental.pallas import tpu_sc as plsc

_NC, _NS, _L = 2, 16, 16
_NW = _NC * _NS
_CH = 128
_CP = pltpu.CompilerParams(use_tc_tiling_on_sc=False)


def _lane_bcast(vec, lane):
    idx = jnp.full((16, 1), lane, jnp.int32)
    dn = lax.GatherDimensionNumbers(
        offset_dims=(), collapsed_slice_dims=(0,), start_index_map=(0,))
    return lax.gather(vec, idx, dn, (1,),
                      mode=lax.GatherScatterMode.PROMISE_IN_BOUNDS)


def _make_deg_pass(NPAD, EPT_CHUNKS):
    EPT = EPT_CHUNKS * _CH
    RPT = NPAD // _NS
    mesh = plsc.VectorSubcoreMesh(core_axis_name="c", subcore_axis_name="s")

    @functools.partial(
        pl.kernel,
        out_type=jax.ShapeDtypeStruct((_NC, NPAD, 16), jnp.float32),
        mesh=mesh,
        compiler_params=_CP,
        scratch_types=[
            pltpu.VMEM((_CH,), jnp.int32),
            pltpu.VMEM((_CH, 16), jnp.float32),
            pltpu.VMEM_SHARED((NPAD, 16), jnp.float32),
        ],
    )
    def deg_pass(dst_hbm, out_hbm, idx_d, ones, acc):
        c = lax.axis_index("c")
        s = lax.axis_index("s")
        tid = c * _NS + s
        zero = jnp.zeros((16,), jnp.float32)
        one = jnp.ones((16,), jnp.float32)

        def zrow(r, _):
            ones[r, pl.ds(0, 16)] = zero
            return 0
        lax.fori_loop(0, _CH, zrow, 0)
        for k in range(RPT // _CH):
            pltpu.sync_copy(ones, acc.at[pl.ds(s * RPT + k * _CH, _CH)])

        def orow(r, _):
            ones[r, pl.ds(0, 16)] = one
            return 0
        lax.fori_loop(0, _CH, orow, 0)
        plsc.subcore_barrier()

        def chunk(ci, _):
            base = tid * EPT + ci * _CH
            pltpu.sync_copy(dst_hbm.at[pl.ds(base, _CH)], idx_d)
            pltpu.sync_copy(ones, acc.at[idx_d], add=True)
            return 0
        lax.fori_loop(0, EPT_CHUNKS, chunk, 0)
        plsc.subcore_barrier()
        for k in range(RPT // _CH):
            r0 = s * RPT + k * _CH
            pltpu.sync_copy(acc.at[pl.ds(r0, _CH)], ones)
            pltpu.sync_copy(ones, out_hbm.at[c, pl.ds(r0, _CH)])

    return deg_pass


_NB = 4


def _make_gcn_pass(NPAD, W, EPT_CHUNKS):
    RPT = NPAD // _NS
    mesh = plsc.VectorSubcoreMesh(core_axis_name="c", subcore_axis_name="s")

    @functools.partial(
        pl.kernel,
        out_type=jax.ShapeDtypeStruct((_NC, NPAD, W), jnp.float32),
        mesh=mesh,
        compiler_params=_CP,
        scratch_types=[
            pltpu.VMEM((EPT_CHUNKS, _CH), jnp.int32),
            pltpu.VMEM((EPT_CHUNKS, _CH), jnp.int32),
            [pltpu.VMEM((_CH, W), jnp.float32)] * _NB,
            pltpu.VMEM_SHARED((NPAD, W), jnp.float32),
            [pltpu.SemaphoreType.DMA] * _NB,
            [pltpu.SemaphoreType.DMA] * _NB,
        ],
    )
    def gcn_pass(table_hbm, src_hbm, dst_hbm, out_hbm, isrc, idst, rows, acc,
                 gsem, ssem):
        c = lax.axis_index("c")
        s = lax.axis_index("s")
        tid = c * _NS + s
        zero = jnp.zeros((16,), jnp.float32)

        def zrow(r, _):
            for k in range(W // 16):
                rows[0][r, pl.ds(k * 16, 16)] = zero
            return 0
        lax.fori_loop(0, _CH, zrow, 0)
        for k in range(RPT // _CH):
            pltpu.sync_copy(rows[0], acc.at[pl.ds(s * RPT + k * _CH, _CH)])
        pltpu.sync_copy(src_hbm.at[tid], isrc)
        pltpu.sync_copy(dst_hbm.at[tid], idst)
        plsc.subcore_barrier()

        gets = {}
        for b in range(min(_NB, EPT_CHUNKS)):
            gets[b] = pltpu.async_copy(table_hbm.at[isrc.at[b]], rows[b],
                                       gsem[b])
        puts = [None] * _NB
        for ci in range(EPT_CHUNKS):
            b = ci % _NB
            gets.pop(ci).wait()
            puts[b] = pltpu.async_copy(rows[b], acc.at[idst.at[ci]], ssem[b],
                                       add=True)
            nxt = ci + _NB
            if nxt < EPT_CHUNKS:
                puts[b].wait()
                puts[b] = None
                gets[nxt] = pltpu.async_copy(table_hbm.at[isrc.at[nxt]],
                                             rows[b], gsem[b])
        for b in range(_NB):
            if puts[b] is not None:
                puts[b].wait()
        plsc.subcore_barrier()
        for k in range(RPT // _CH):
            r0 = s * RPT + k * _CH
            pltpu.sync_copy(acc.at[pl.ds(r0, _CH)], rows[0])
            pltpu.sync_copy(rows[0], out_hbm.at[c, pl.ds(r0, _CH)])

    return gcn_pass


_NBG = 3


def _make_gat_pass(NPAD, EPT_CHUNKS):
    RPT = NPAD // _NS
    mesh = plsc.VectorSubcoreMesh(core_axis_name="c", subcore_axis_name="s")

    @functools.partial(
        pl.kernel,
        out_type=(
            jax.ShapeDtypeStruct((_NC, NPAD, 64), jnp.float32),
            jax.ShapeDtypeStruct((_NC, NPAD, 16), jnp.float32),
        ),
        mesh=mesh,
        compiler_params=_CP,
        scratch_types=[
            pltpu.VMEM((EPT_CHUNKS, _CH), jnp.int32),
            pltpu.VMEM((EPT_CHUNKS, _CH), jnp.int32),
            [pltpu.VMEM((_CH, 64), jnp.float32)] * _NBG,
            [pltpu.VMEM((_CH, 16), jnp.float32)] * _NBG,
            [pltpu.VMEM((_CH, 16), jnp.float32)] * _NBG,
            [pltpu.VMEM((_CH, 16), jnp.float32)] * _NBG,
            pltpu.VMEM_SHARED((NPAD, 64), jnp.float32),
            pltpu.VMEM_SHARED((NPAD, 16), jnp.float32),
            [pltpu.SemaphoreType.DMA] * _NBG,
            [pltpu.SemaphoreType.DMA] * _NBG,
            [pltpu.SemaphoreType.DMA] * _NBG,
            [pltpu.SemaphoreType.DMA] * _NBG,
        ],
    )
    def gat_pass(table_hbm, als_hbm, ald_hbm, src_hbm, dst_hbm, num_hbm, den_hbm,
                 isrc, idst, rows, ars, ard, wbuf, num_acc, den_acc,
                 rsem, asem, nsem, dsem):
        c = lax.axis_index("c")
        s = lax.axis_index("s")
        tid = c * _NS + s
        zero = jnp.zeros((16,), jnp.float32)

        def zrow(r, _):
            for k in range(4):
                rows[0][r, pl.ds(k * 16, 16)] = zero
            wbuf[0][r, pl.ds(0, 16)] = zero
            return 0
        lax.fori_loop(0, _CH, zrow, 0)
        for k in range(RPT // _CH):
            r0 = s * RPT + k * _CH
            pltpu.sync_copy(rows[0], num_acc.at[pl.ds(r0, _CH)])
            pltpu.sync_copy(wbuf[0], den_acc.at[pl.ds(r0, _CH)])
        pltpu.sync_copy(src_hbm.at[tid], isrc)
        pltpu.sync_copy(dst_hbm.at[tid], idst)
        plsc.subcore_barrier()

        def gathers(ci, b):
            return (
                pltpu.async_copy(table_hbm.at[isrc.at[ci]], rows[b], rsem[b]),
                pltpu.async_copy(als_hbm.at[isrc.at[ci]], ars[b], asem[b]),
                pltpu.async_copy(ald_hbm.at[idst.at[ci]], ard[b], asem[b]),
            )

        gets = {}
        for b in range(min(_NBG, EPT_CHUNKS)):
            gets[b] = gathers(b, b)
        putsN = [None] * _NBG
        putsD = [None] * _NBG
        for ci in range(EPT_CHUNKS):
            b = ci % _NBG
            g1, g2, g3 = gets.pop(ci)
            g2.wait()
            g3.wait()
            g1.wait()
            rows_b, ars_b, ard_b, wbuf_b = rows[b], ars[b], ard[b], wbuf[b]

            def edge(r, _):
                ev = ars_b[r, pl.ds(0, 16)] + ard_b[r, pl.ds(0, 16)]
                ev = jnp.where(ev > 0, ev, 0.2 * ev)
                wv = jnp.exp(ev)
                wbuf_b[r, pl.ds(0, 16)] = wv
                for h in range(4):
                    bb = _lane_bcast(wv, h)
                    rows_b[r, pl.ds(h * 16, 16)] = rows_b[r, pl.ds(h * 16, 16)] * bb
                return 0
            lax.fori_loop(0, _CH, edge, 0)
            putsN[b] = pltpu.async_copy(rows[b], num_acc.at[idst.at[ci]],
                                        nsem[b], add=True)
            putsD[b] = pltpu.async_copy(wbuf[b], den_acc.at[idst.at[ci]],
                                        dsem[b], add=True)
            nxt = ci + _NBG
            if nxt < EPT_CHUNKS:
                putsN[b].wait()
                putsD[b].wait()
                putsN[b] = putsD[b] = None
                gets[nxt] = gathers(nxt, b)
        for b in range(_NBG):
            if putsN[b] is not None:
                putsN[b].wait()
                putsD[b].wait()
        plsc.subcore_barrier()
        for k in range(RPT // _CH):
            r0 = s * RPT + k * _CH
            pltpu.sync_copy(num_acc.at[pl.ds(r0, _CH)], rows[0])
            pltpu.sync_copy(rows[0], num_hbm.at[c, pl.ds(r0, _CH)])
            pltpu.sync_copy(den_acc.at[pl.ds(r0, _CH)], wbuf[0])
            pltpu.sync_copy(wbuf[0], den_hbm.at[c, pl.ds(r0, _CH)])

    return gat_pass


_BR = 2000


def _tc_layer1(x, W1, d0, d1):
    N, Din = x.shape
    Hid = W1.shape[1]
    grid = N // _BR

    def body(x_ref, w_ref, d0_ref, d1_ref, t_ref, dis_ref):
        deg = d0_ref[...] + d1_ref[...] + 1.0
        dis = lax.rsqrt(deg)
        dis_ref[...] = dis
        t_ref[...] = jnp.dot(x_ref[...], w_ref[...],
                             preferred_element_type=jnp.float32) * dis

    return pl.pallas_call(
        body,
        grid=(grid,),
        in_specs=[
            pl.BlockSpec((_BR, Din), lambda i: (i, 0)),
            pl.BlockSpec((Din, Hid), lambda i: (0, 0)),
            pl.BlockSpec((_BR, 1), lambda i: (i, 0)),
            pl.BlockSpec((_BR, 1), lambda i: (i, 0)),
        ],
        out_specs=[
            pl.BlockSpec((_BR, Hid), lambda i: (i, 0)),
            pl.BlockSpec((_BR, 1), lambda i: (i, 0)),
        ],
        out_shape=[
            jax.ShapeDtypeStruct((N, Hid), jnp.float32),
            jax.ShapeDtypeStruct((N, 1), jnp.float32),
        ],
    )(x, W1, d0, d1)


def _tc_mid(p0, p1, t, dis, b, Wn):
    N, W = t.shape
    W2 = Wn.shape[1]
    grid = N // _BR

    def body(p0_ref, p1_ref, t_ref, dis_ref, b_ref, w_ref, out_ref):
        dis = dis_ref[...]
        h = dis * (p0_ref[...] + p1_ref[...] + t_ref[...]) + b_ref[...]
        h = jnp.maximum(h, 0.0)
        out_ref[...] = jnp.dot(h, w_ref[...],
                               preferred_element_type=jnp.float32) * dis

    return pl.pallas_call(
        body,
        grid=(grid,),
        in_specs=[
            pl.BlockSpec((_BR, W), lambda i: (i, 0)),
            pl.BlockSpec((_BR, W), lambda i: (i, 0)),
            pl.BlockSpec((_BR, W), lambda i: (i, 0)),
            pl.BlockSpec((_BR, 1), lambda i: (i, 0)),
            pl.BlockSpec((1, W), lambda i: (0, 0)),
            pl.BlockSpec((W, W2), lambda i: (0, 0)),
        ],
        out_specs=pl.BlockSpec((_BR, W2), lambda i: (i, 0)),
        out_shape=jax.ShapeDtypeStruct((N, W2), jnp.float32),
    )(p0, p1, t, dis, b, Wn)


def _tc_mid3(p0, p1, t, dis, b3, Wa, a_sf, a_df):
    N, W = t.shape
    HC = Wa.shape[1]
    grid = N // _BR

    def body(p0_ref, p1_ref, t_ref, dis_ref, b_ref, wa_ref, as_ref, ad_ref,
             hg_ref, als_ref, ald_ref, ws_ref):
        dis = dis_ref[...]
        h3 = dis * (p0_ref[...] + p1_ref[...] + t_ref[...]) + b_ref[...]
        h3 = jnp.maximum(h3, 0.0)
        hg = jnp.dot(h3, wa_ref[...], preferred_element_type=jnp.float32)
        hg_ref[...] = hg
        al_s = (hg * as_ref[...]).reshape(_BR, 4, 16).sum(-1)
        al_d = (hg * ad_ref[...]).reshape(_BR, 4, 16).sum(-1)
        z12 = jnp.zeros((_BR, 12), jnp.float32)
        als_ref[...] = jnp.concatenate([al_s, z12], axis=1)
        ald_ref[...] = jnp.concatenate([al_d, z12], axis=1)
        es = al_s + al_d
        es = jnp.where(es > 0, es, 0.2 * es)
        ws_ref[...] = jnp.exp(es)

    return pl.pallas_call(
        body,
        grid=(grid,),
        in_specs=[
            pl.BlockSpec((_BR, W), lambda i: (i, 0)),
            pl.BlockSpec((_BR, W), lambda i: (i, 0)),
            pl.BlockSpec((_BR, W), lambda i: (i, 0)),
            pl.BlockSpec((_BR, 1), lambda i: (i, 0)),
            pl.BlockSpec((1, W), lambda i: (0, 0)),
            pl.BlockSpec((W, HC), lambda i: (0, 0)),
            pl.BlockSpec((1, HC), lambda i: (0, 0)),
            pl.BlockSpec((1, HC), lambda i: (0, 0)),
        ],
        out_specs=[
            pl.BlockSpec((_BR, HC), lambda i: (i, 0)),
            pl.BlockSpec((_BR, 16), lambda i: (i, 0)),
            pl.BlockSpec((_BR, 16), lambda i: (i, 0)),
            pl.BlockSpec((_BR, 4), lambda i: (i, 0)),
        ],
        out_shape=[
            jax.ShapeDtypeStruct((N, HC), jnp.float32),
            jax.ShapeDtypeStruct((N, 16), jnp.float32),
            jax.ShapeDtypeStruct((N, 16), jnp.float32),
            jax.ShapeDtypeStruct((N, 4), jnp.float32),
        ],
    )(p0, p1, t, dis, b3, Wa, a_sf, a_df)


def _tc_final(np0, np1, d0, d1, hg, ws, b_att):
    N, HC = hg.shape
    grid = N // _BR

    def body(np0_ref, np1_ref, d0_ref, d1_ref, hg_ref, ws_ref, b_ref, acc_ref):
        i = pl.program_id(0)
        ws = ws_ref[...]
        wrep = jnp.broadcast_to(ws[:, :, None], (_BR, 4, 16)).reshape(_BR, 64)
        num = np0_ref[...] + np1_ref[...] + wrep * hg_ref[...]
        den4 = d0_ref[...][:, :4] + d1_ref[...][:, :4] + ws
        denrep = jnp.broadcast_to(den4[:, :, None], (_BR, 4, 16)).reshape(_BR, 64)
        h_att = jnp.maximum(num / (denrep + 1e-16) + b_ref[...], 0.0)
        part = jnp.sum(h_att, axis=0, keepdims=True)

        @pl.when(i == 0)
        def _():
            acc_ref[...] = jnp.zeros_like(acc_ref)
        acc_ref[...] += part

    return pl.pallas_call(
        body,
        grid=(grid,),
        in_specs=[
            pl.BlockSpec((_BR, HC), lambda i: (i, 0)),
            pl.BlockSpec((_BR, HC), lambda i: (i, 0)),
            pl.BlockSpec((_BR, 16), lambda i: (i, 0)),
            pl.BlockSpec((_BR, 16), lambda i: (i, 0)),
            pl.BlockSpec((_BR, HC), lambda i: (i, 0)),
            pl.BlockSpec((_BR, 4), lambda i: (i, 0)),
            pl.BlockSpec((1, HC), lambda i: (0, 0)),
        ],
        out_specs=pl.BlockSpec((1, HC), lambda i: (0, 0)),
        out_shape=jax.ShapeDtypeStruct((1, HC), jnp.float32),
    )(np0, np1, d0, d1, hg, ws, b_att)


def _tc_mlp(colsum, n_nodes, Wc1p, bc1p, Wc2p, bc2p):
    def body(cs_ref, w1_ref, b1_ref, w2_ref, b2_ref, out_ref):
        pooled = cs_ref[...] * (1.0 / n_nodes)
        z = jnp.maximum(jnp.dot(pooled, w1_ref[...],
                                preferred_element_type=jnp.float32)
                        + b1_ref[...], 0.0)
        out_ref[...] = jnp.dot(z, w2_ref[...],
                               preferred_element_type=jnp.float32) + b2_ref[...]

    return pl.pallas_call(
        body,
        out_shape=jax.ShapeDtypeStruct((1, 128), jnp.float32),
    )(colsum, Wc1p, bc1p, Wc2p, bc2p)


def kernel(x, edge_index, W1, b1, W2, b2, W3, b3, Wa, a_src, a_dst, b_att,
           Wc1, bc1, Wc2, bc2):
    N = x.shape[0]
    E = edge_index.shape[1]
    NPAD = 10240
    EPT_CHUNKS = -(-E // (_NW * _CH))
    EPAD = EPT_CHUNKS * _CH * _NW

    src = edge_index[0]
    dst = edge_index[1]
    pad_idx = jnp.full((EPAD - E,), N, jnp.int32)
    srcp = jnp.concatenate([src, pad_idx])
    dstp = jnp.concatenate([dst, pad_idx])
    srcp3 = srcp.reshape(_NW, EPT_CHUNKS, _CH)
    dstp3 = dstp.reshape(_NW, EPT_CHUNKS, _CH)

    def padN(a):
        return jnp.pad(a, ((0, NPAD - N), (0, 0)))

    degp = _make_deg_pass(NPAD, EPT_CHUNKS)(dstp)
    d0 = degp[0, :N, 0:1]
    d1 = degp[1, :N, 0:1]

    t1, dis = _tc_layer1(x, W1, d0, d1)
    p = _make_gcn_pass(NPAD, 64, EPT_CHUNKS)(padN(t1), srcp3, dstp3)
    t2 = _tc_mid(p[0, :N], p[1, :N], t1, dis, b1.reshape(1, -1), W2)
    p = _make_gcn_pass(NPAD, 64, EPT_CHUNKS)(padN(t2), srcp3, dstp3)
    t3 = _tc_mid(p[0, :N], p[1, :N], t2, dis, b2.reshape(1, -1), W3)
    p = _make_gcn_pass(NPAD, 32, EPT_CHUNKS)(padN(t3), srcp3, dstp3)

    a_sf = a_src.reshape(1, -1)
    a_df = a_dst.reshape(1, -1)
    hg, als16, ald16, ws = _tc_mid3(p[0, :N], p[1, :N], t3, dis,
                                    b3.reshape(1, -1), Wa, a_sf, a_df)

    nump, denp = _make_gat_pass(NPAD, EPT_CHUNKS)(
        padN(hg), padN(als16), padN(ald16), srcp3, dstp3)

    colsum = _tc_final(nump[0, :N], nump[1, :N], denp[0, :N], denp[1, :N],
                       hg, ws, b_att.reshape(1, -1))
    bc1p = bc1.reshape(1, -1)
    Wc2p = jnp.pad(Wc2, ((0, 0), (0, 126)))
    bc2p = jnp.pad(bc2.reshape(1, -1), ((0, 0), (0, 126)))
    outp = _tc_mlp(colsum, float(N), Wc1, bc1p, Wc2p, bc2p)
    return outp[:, :2]
